# Initial kernel scaffold; baseline (speedup 1.0000x reference)
import jax
import jax.numpy as jnp
from jax.experimental import pallas as pl


def kernel(x, weight, bias):
    raise NotImplementedError("write your pallas kernel here")


# speedup vs baseline: 1.0446x; 1.0446x over previous
import jax
import jax.numpy as jnp
from jax.experimental import pallas as pl
from jax.experimental.pallas import tpu as pltpu

_VMEM_BUDGET = (64 * 1024 * 1024 * 3) // 4


def _linear_kernel(x_ref, w_ref, b_ref, o_ref):
    acc = jax.lax.dot_general(
        x_ref[...], w_ref[...],
        dimension_numbers=(((1,), (1,)), ((), ())),
        preferred_element_type=jnp.float32)
    o_ref[...] = (acc + b_ref[...].astype(jnp.float32)).astype(o_ref.dtype)


def kernel(x, weight, bias):
    n, k = x.shape
    h = weight.shape[0]
    out_dtype = x.dtype

    tm = min(1024, n)
    grid = (pl.cdiv(n, tm),)

    b_row = bias.reshape(1, h)

    bytes_accessed = (x.size * x.dtype.itemsize
                      + weight.size * weight.dtype.itemsize
                      + bias.size * bias.dtype.itemsize
                      + n * h * jnp.dtype(out_dtype).itemsize)

    return pl.pallas_call(
        _linear_kernel,
        out_shape=jax.ShapeDtypeStruct((n, h), out_dtype),
        grid=grid,
        in_specs=[
            pl.BlockSpec((tm, k), lambda i: (i, 0)),
            pl.BlockSpec((h, k), lambda i: (0, 0)),
            pl.BlockSpec((1, h), lambda i: (0, 0)),
        ],
        out_specs=pl.BlockSpec((tm, h), lambda i: (i, 0)),
        compiler_params=pltpu.CompilerParams(
            dimension_semantics=("parallel",),
            vmem_limit_bytes=_VMEM_BUDGET,
        ),
        cost_estimate=pl.CostEstimate(
            flops=2 * n * h * k,
            bytes_accessed=bytes_accessed,
            transcendentals=0),
    )(x, weight, b_row)
